# baseline (speedup 1.0000x reference)
import functools

import jax
import jax.numpy as jnp
from jax import lax
from jax.experimental import pallas as pl
from jax.experimental.pallas import tpu as pltpu
from jax.experimental.pallas import tpu_sc as plsc

N = 10000
E = 320000
D_IN = 128
D2 = 48

NC = 2
NS = 16
NW = NC * NS
CHUNK = 128
NCH_REAL = E // CHUNK
NCH = 2560
CPW = NCH // NW
SUB = 16
NBLK = CPW // SUB
DUMP = N
NROW = 10016
RPT = NROW // NS
DEG_PAD = 10240
RPD = DEG_PAD // NS


def _make_edge_agg(d, with_deg, stage_t, ring):
    mesh = plsc.VectorSubcoreMesh(core_axis_name="c", subcore_axis_name="s")

    out_type = [jax.ShapeDtypeStruct((NC, NROW, d), jnp.float32)]
    if with_deg:
        out_type.append(jax.ShapeDtypeStruct((NC * DEG_PAD,), jnp.float32))

    scratch = [
        pltpu.VMEM((SUB, 2, CHUNK), jnp.int32),
        pltpu.VMEM((ring, CHUNK, d), jnp.float32),
        pltpu.VMEM((CHUNK,), jnp.float32),
        pltpu.VMEM_SHARED((NROW, d), jnp.float32),
        pltpu.VMEM_SHARED((N, d) if stage_t else (8, d), jnp.float32),
        pltpu.VMEM_SHARED((DEG_PAD if with_deg else 16,), jnp.float32),
        pltpu.SemaphoreType.DMA,
        pltpu.SemaphoreType.DMA,
        pltpu.SemaphoreType.DMA,
    ]

    def body(t_hbm, ei_hbm, za_hbm, zd_hbm, out_hbm, deg_hbm,
             idxb_v, rows_v, ones_v, acc, tstage, dacc, sem_s, sem_g, sem_d):
        c = lax.axis_index("c")
        s = lax.axis_index("s")
        w = c * NS + s

        pltpu.sync_copy(za_hbm, acc.at[pl.ds(s * RPT, RPT)])
        if with_deg:
            for i in range(CHUNK // 16):
                ones_v[pl.ds(i * 16, 16)] = jnp.ones((16,), jnp.float32)
            pltpu.sync_copy(zd_hbm, dacc.at[pl.ds(s * RPD, RPD)])
        if stage_t:
            npt = N // NS
            pltpu.sync_copy(t_hbm.at[pl.ds(s * npt, npt)],
                            tstage.at[pl.ds(s * npt, npt)])
        gsrc = tstage if stage_t else t_hbm
        plsc.subcore_barrier()

        def gissue(j):
            pltpu.async_copy(gsrc.at[idxb_v.at[j, 0]],
                             rows_v.at[lax.rem(j, ring)], sem_g)

        def gwait():
            pltpu.make_async_copy(gsrc.at[idxb_v.at[0, 0]], rows_v.at[0],
                                  sem_g).wait()

        def swait():
            pltpu.make_async_copy(rows_v.at[0], acc.at[idxb_v.at[0, 1]],
                                  sem_s).wait()

        def dwait():
            pltpu.make_async_copy(ones_v, dacc.at[idxb_v.at[0, 1]],
                                  sem_d).wait()

        chunk0 = w * CPW

        def block(b, carry):
            row0 = chunk0 + b * SUB
            pltpu.sync_copy(ei_hbm.at[pl.ds(row0, SUB)], idxb_v)
            gissue(0)
            gissue(1)

            def step(j, carry2):
                gwait()
                pltpu.async_copy(rows_v.at[lax.rem(j, ring)],
                                 acc.at[idxb_v.at[j, 1]], sem_s, add=True)
                if with_deg:
                    pltpu.async_copy(ones_v, dacc.at[idxb_v.at[j, 1]],
                                     sem_d, add=True)

                @pl.when(j + 2 < SUB)
                def _():
                    @pl.when(j >= 1)
                    def _():
                        swait()
                        if with_deg:
                            dwait()

                    gissue(j + 2)

                return carry2

            lax.fori_loop(0, SUB, step, 0)
            for _ in range(3):
                swait()
                if with_deg:
                    dwait()
            return carry

        lax.fori_loop(0, NBLK, block, 0)
        plsc.subcore_barrier()
        pltpu.sync_copy(acc.at[pl.ds(s * RPT, RPT)],
                        out_hbm.at[c, pl.ds(s * RPT, RPT)])
        if with_deg:
            pltpu.sync_copy(dacc.at[pl.ds(s * RPD, RPD)],
                            deg_hbm.at[pl.ds(c * DEG_PAD + s * RPD, RPD)])

    if with_deg:
        def agg(t_hbm, ei_hbm, za_hbm, zd_hbm, out_hbm, deg_hbm,
                idxb_v, rows_v, ones_v, acc, tstage, dacc,
                sem_s, sem_g, sem_d):
            body(t_hbm, ei_hbm, za_hbm, zd_hbm, out_hbm, deg_hbm,
                 idxb_v, rows_v, ones_v, acc, tstage, dacc,
                 sem_s, sem_g, sem_d)
    else:
        def agg(t_hbm, ei_hbm, za_hbm, zd_hbm, out_hbm,
                idxb_v, rows_v, ones_v, acc, tstage, dacc,
                sem_s, sem_g, sem_d):
            body(t_hbm, ei_hbm, za_hbm, zd_hbm, out_hbm, None,
                 idxb_v, rows_v, ones_v, acc, tstage, dacc,
                 sem_s, sem_g, sem_d)

    return functools.partial(
        pl.kernel,
        mesh=mesh,
        out_type=out_type if with_deg else out_type[0],
        scratch_types=scratch,
        compiler_params=pltpu.CompilerParams(use_tc_tiling_on_sc=False),
    )(agg)


_agg_d1 = _make_edge_agg(D_IN, with_deg=True, stage_t=False, ring=2)
_agg_d2 = _make_edge_agg(D2, with_deg=False, stage_t=True, ring=3)

_BN = 1000


def _tc_mid(x, p, invd, ws1, b1r, wn1, wn2p, ws2):
    def body(x_ref, p_ref, invd_ref, ws1_ref, b1_ref, wn1_ref, wn2_ref,
             ws2_ref, h1_ref, t2_ref, s2_ref):
        mean = (p_ref[0] + p_ref[1]) * invd_ref[...]
        h1 = jnp.maximum(
            jnp.dot(x_ref[...], ws1_ref[...],
                    preferred_element_type=jnp.float32)
            + jnp.dot(mean, wn1_ref[...], preferred_element_type=jnp.float32)
            + b1_ref[...],
            0.0)
        h1_ref[...] = h1
        t2_ref[...] = jnp.dot(h1, wn2_ref[...],
                              preferred_element_type=jnp.float32)
        s2_ref[...] = jnp.dot(h1, ws2_ref[...],
                              preferred_element_type=jnp.float32)

    return pl.pallas_call(
        body,
        grid=(N // _BN,),
        in_specs=[
            pl.BlockSpec((_BN, D_IN), lambda i: (i, 0)),
            pl.BlockSpec((NC, _BN, D_IN), lambda i: (0, i, 0)),
            pl.BlockSpec((_BN, 1), lambda i: (i, 0)),
            pl.BlockSpec((D_IN, D_IN), lambda i: (0, 0)),
            pl.BlockSpec((1, D_IN), lambda i: (0, 0)),
            pl.BlockSpec((D_IN, D_IN), lambda i: (0, 0)),
            pl.BlockSpec((D_IN, D2), lambda i: (0, 0)),
            pl.BlockSpec((D_IN, 40), lambda i: (0, 0)),
        ],
        out_specs=[
            pl.BlockSpec((_BN, D_IN), lambda i: (i, 0)),
            pl.BlockSpec((_BN, D2), lambda i: (i, 0)),
            pl.BlockSpec((_BN, 40), lambda i: (i, 0)),
        ],
        out_shape=[
            jax.ShapeDtypeStruct((N, D_IN), jnp.float32),
            jax.ShapeDtypeStruct((N, D2), jnp.float32),
            jax.ShapeDtypeStruct((N, 40), jnp.float32),
        ],
    )(x, p, invd, ws1, b1r, wn1, wn2p, ws2)


def _tc_post(s2, q, invd, b2r):
    def body(s2_ref, q_ref, invd_ref, b2_ref, o_ref):
        aggq = q_ref[0] + q_ref[1]
        o_ref[...] = s2_ref[...] + aggq[:, :40] * invd_ref[...] + b2_ref[...]

    return pl.pallas_call(
        body,
        grid=(N // _BN,),
        in_specs=[
            pl.BlockSpec((_BN, 40), lambda i: (i, 0)),
            pl.BlockSpec((NC, _BN, D2), lambda i: (0, i, 0)),
            pl.BlockSpec((_BN, 1), lambda i: (i, 0)),
            pl.BlockSpec((1, 40), lambda i: (0, 0)),
        ],
        out_specs=pl.BlockSpec((_BN, 40), lambda i: (i, 0)),
        out_shape=jax.ShapeDtypeStruct((N, 40), jnp.float32),
    )(s2, q, invd, b2r)


def kernel(inputs, edge_index, W_self1, W_neigh1, b1, W_self2, W_neigh2, b2):
    ei = edge_index.reshape(2, NCH_REAL, CHUNK)
    pad = jnp.concatenate(
        [jnp.zeros((1, NCH - NCH_REAL, CHUNK), jnp.int32),
         jnp.full((1, NCH - NCH_REAL, CHUNK), DUMP, jnp.int32)], axis=0)
    ei3 = jnp.concatenate([ei, pad], axis=1).transpose(1, 0, 2)

    wn2p = jnp.pad(W_neigh2, ((0, 0), (0, D2 - 40)))
    b1r = b1.reshape(1, D_IN)
    b2r = b2.reshape(1, 40)
    za1 = jnp.zeros((RPT, D_IN), jnp.float32)
    za2 = jnp.zeros((RPT, D2), jnp.float32)
    zd = jnp.zeros((RPD,), jnp.float32)

    p, degs = _agg_d1(inputs, ei3, za1, zd)
    deg = degs[:N] + degs[DEG_PAD:DEG_PAD + N]
    invd = (1.0 / jnp.clip(deg, 1.0, None)).reshape(N, 1)
    h1, t2p, s2 = _tc_mid(inputs, p, invd, W_self1, b1r, W_neigh1,
                          wn2p, W_self2)
    q = _agg_d2(t2p, ei3, za2, zd)
    out = _tc_post(s2, q, invd, b2r)
    return (out, h1, out, h1)


# speedup vs baseline: 2.5295x; 2.5295x over previous
import functools

import jax
import jax.numpy as jnp
from jax import lax
from jax.experimental import pallas as pl
from jax.experimental.pallas import tpu as pltpu
from jax.experimental.pallas import tpu_sc as plsc

N = 10000
E = 320000
D_IN = 128
D2 = 48

NC = 2
NS = 16
NW = NC * NS
CHUNK = 128
NCH_REAL = E // CHUNK
NCH = 2560
CPW = NCH // NW
SUB = 16
NBLK = CPW // SUB
NROW = N + CHUNK
RPT = NROW // NS
DEG_PAD = 10240
RPD = DEG_PAD // NS


def _make_edge_agg(d, with_deg, stage_t, ring):
    mesh = plsc.VectorSubcoreMesh(core_axis_name="c", subcore_axis_name="s")

    out_type = [jax.ShapeDtypeStruct((NC, NROW, d), jnp.float32)]
    if with_deg:
        out_type.append(jax.ShapeDtypeStruct((NC * DEG_PAD,), jnp.float32))

    scratch = [
        pltpu.VMEM((SUB, 2, CHUNK), jnp.int32),
        pltpu.VMEM((ring, CHUNK, d), jnp.float32),
        pltpu.VMEM((CHUNK,), jnp.float32),
        pltpu.VMEM_SHARED((NROW, d), jnp.float32),
        pltpu.VMEM_SHARED((N, d) if stage_t else (8, d), jnp.float32),
        pltpu.VMEM_SHARED((DEG_PAD if with_deg else 16,), jnp.float32),
        pltpu.SemaphoreType.DMA,
        pltpu.SemaphoreType.DMA,
        pltpu.SemaphoreType.DMA,
    ]

    def body(t_hbm, ei_hbm, za_hbm, zd_hbm, out_hbm, deg_hbm,
             idxb_v, rows_v, ones_v, acc, tstage, dacc, sem_s, sem_g, sem_d):
        c = lax.axis_index("c")
        s = lax.axis_index("s")
        w = c * NS + s

        pltpu.sync_copy(za_hbm, acc.at[pl.ds(s * RPT, RPT)])
        if with_deg:
            for i in range(CHUNK // 16):
                ones_v[pl.ds(i * 16, 16)] = jnp.ones((16,), jnp.float32)
            pltpu.sync_copy(zd_hbm, dacc.at[pl.ds(s * RPD, RPD)])
        if stage_t:
            npt = N // NS
            pltpu.sync_copy(t_hbm.at[pl.ds(s * npt, npt)],
                            tstage.at[pl.ds(s * npt, npt)])
        gsrc = tstage if stage_t else t_hbm
        plsc.subcore_barrier()

        def gissue(j):
            pltpu.async_copy(gsrc.at[idxb_v.at[j, 0]],
                             rows_v.at[lax.rem(j, ring)], sem_g)

        def gwait():
            pltpu.make_async_copy(gsrc.at[idxb_v.at[0, 0]], rows_v.at[0],
                                  sem_g).wait()

        def swait():
            pltpu.make_async_copy(rows_v.at[0], acc.at[idxb_v.at[0, 1]],
                                  sem_s).wait()

        def dwait():
            pltpu.make_async_copy(ones_v, dacc.at[idxb_v.at[0, 1]],
                                  sem_d).wait()

        chunk0 = w * CPW

        def block(b, carry):
            row0 = chunk0 + b * SUB
            pltpu.sync_copy(ei_hbm.at[pl.ds(row0, SUB)], idxb_v)
            gissue(0)
            gissue(1)

            def step(j, carry2):
                gwait()
                pltpu.async_copy(rows_v.at[lax.rem(j, ring)],
                                 acc.at[idxb_v.at[j, 1]], sem_s, add=True)
                if with_deg:
                    pltpu.async_copy(ones_v, dacc.at[idxb_v.at[j, 1]],
                                     sem_d, add=True)

                @pl.when(j + 2 < SUB)
                def _():
                    @pl.when(j >= 1)
                    def _():
                        swait()
                        if with_deg:
                            dwait()

                    gissue(j + 2)

                return carry2

            lax.fori_loop(0, SUB, step, 0)
            for _ in range(3):
                swait()
                if with_deg:
                    dwait()
            return carry

        lax.fori_loop(0, NBLK, block, 0)
        plsc.subcore_barrier()
        pltpu.sync_copy(acc.at[pl.ds(s * RPT, RPT)],
                        out_hbm.at[c, pl.ds(s * RPT, RPT)])
        if with_deg:
            pltpu.sync_copy(dacc.at[pl.ds(s * RPD, RPD)],
                            deg_hbm.at[pl.ds(c * DEG_PAD + s * RPD, RPD)])

    if with_deg:
        def agg(t_hbm, ei_hbm, za_hbm, zd_hbm, out_hbm, deg_hbm,
                idxb_v, rows_v, ones_v, acc, tstage, dacc,
                sem_s, sem_g, sem_d):
            body(t_hbm, ei_hbm, za_hbm, zd_hbm, out_hbm, deg_hbm,
                 idxb_v, rows_v, ones_v, acc, tstage, dacc,
                 sem_s, sem_g, sem_d)
    else:
        def agg(t_hbm, ei_hbm, za_hbm, zd_hbm, out_hbm,
                idxb_v, rows_v, ones_v, acc, tstage, dacc,
                sem_s, sem_g, sem_d):
            body(t_hbm, ei_hbm, za_hbm, zd_hbm, out_hbm, None,
                 idxb_v, rows_v, ones_v, acc, tstage, dacc,
                 sem_s, sem_g, sem_d)

    return functools.partial(
        pl.kernel,
        mesh=mesh,
        out_type=out_type if with_deg else out_type[0],
        scratch_types=scratch,
        compiler_params=pltpu.CompilerParams(use_tc_tiling_on_sc=False),
    )(agg)


_agg_d1 = _make_edge_agg(D_IN, with_deg=True, stage_t=False, ring=2)
_agg_d2 = _make_edge_agg(D2, with_deg=False, stage_t=True, ring=3)

_BN = 1000


def _tc_mid(x, p, invd, ws1, b1r, wn1, wn2p, ws2):
    def body(x_ref, p_ref, invd_ref, ws1_ref, b1_ref, wn1_ref, wn2_ref,
             ws2_ref, h1_ref, t2_ref, s2_ref):
        mean = (p_ref[0] + p_ref[1]) * invd_ref[...]
        h1 = jnp.maximum(
            jnp.dot(x_ref[...], ws1_ref[...],
                    preferred_element_type=jnp.float32)
            + jnp.dot(mean, wn1_ref[...], preferred_element_type=jnp.float32)
            + b1_ref[...],
            0.0)
        h1_ref[...] = h1
        t2_ref[...] = jnp.dot(h1, wn2_ref[...],
                              preferred_element_type=jnp.float32)
        s2_ref[...] = jnp.dot(h1, ws2_ref[...],
                              preferred_element_type=jnp.float32)

    return pl.pallas_call(
        body,
        grid=(N // _BN,),
        in_specs=[
            pl.BlockSpec((_BN, D_IN), lambda i: (i, 0)),
            pl.BlockSpec((NC, _BN, D_IN), lambda i: (0, i, 0)),
            pl.BlockSpec((_BN, 1), lambda i: (i, 0)),
            pl.BlockSpec((D_IN, D_IN), lambda i: (0, 0)),
            pl.BlockSpec((1, D_IN), lambda i: (0, 0)),
            pl.BlockSpec((D_IN, D_IN), lambda i: (0, 0)),
            pl.BlockSpec((D_IN, D2), lambda i: (0, 0)),
            pl.BlockSpec((D_IN, 40), lambda i: (0, 0)),
        ],
        out_specs=[
            pl.BlockSpec((_BN, D_IN), lambda i: (i, 0)),
            pl.BlockSpec((_BN, D2), lambda i: (i, 0)),
            pl.BlockSpec((_BN, 40), lambda i: (i, 0)),
        ],
        out_shape=[
            jax.ShapeDtypeStruct((N, D_IN), jnp.float32),
            jax.ShapeDtypeStruct((N, D2), jnp.float32),
            jax.ShapeDtypeStruct((N, 40), jnp.float32),
        ],
    )(x, p, invd, ws1, b1r, wn1, wn2p, ws2)


def _tc_post(s2, q, invd, b2r):
    def body(s2_ref, q_ref, invd_ref, b2_ref, o_ref):
        aggq = q_ref[0] + q_ref[1]
        o_ref[...] = s2_ref[...] + aggq[:, :40] * invd_ref[...] + b2_ref[...]

    return pl.pallas_call(
        body,
        grid=(N // _BN,),
        in_specs=[
            pl.BlockSpec((_BN, 40), lambda i: (i, 0)),
            pl.BlockSpec((NC, _BN, D2), lambda i: (0, i, 0)),
            pl.BlockSpec((_BN, 1), lambda i: (i, 0)),
            pl.BlockSpec((1, 40), lambda i: (0, 0)),
        ],
        out_specs=pl.BlockSpec((_BN, 40), lambda i: (i, 0)),
        out_shape=jax.ShapeDtypeStruct((N, 40), jnp.float32),
    )(s2, q, invd, b2r)


def kernel(inputs, edge_index, W_self1, W_neigh1, b1, W_self2, W_neigh2, b2):
    ei = edge_index.reshape(2, NCH_REAL, CHUNK)
    lane = jnp.arange(CHUNK, dtype=jnp.int32)
    pad = jnp.stack([jnp.broadcast_to(lane, (NCH - NCH_REAL, CHUNK)),
                     jnp.broadcast_to(N + lane, (NCH - NCH_REAL, CHUNK))])
    ei3 = jnp.concatenate([ei, pad], axis=1).transpose(1, 0, 2)

    wn2p = jnp.pad(W_neigh2, ((0, 0), (0, D2 - 40)))
    b1r = b1.reshape(1, D_IN)
    b2r = b2.reshape(1, 40)
    za1 = jnp.zeros((RPT, D_IN), jnp.float32)
    za2 = jnp.zeros((RPT, D2), jnp.float32)
    zd = jnp.zeros((RPD,), jnp.float32)

    p, degs = _agg_d1(inputs, ei3, za1, zd)
    deg = degs[:N] + degs[DEG_PAD:DEG_PAD + N]
    invd = (1.0 / jnp.clip(deg, 1.0, None)).reshape(N, 1)
    h1, t2p, s2 = _tc_mid(inputs, p, invd, W_self1, b1r, W_neigh1,
                          wn2p, W_self2)
    q = _agg_d2(t2p, ei3, za2, zd)
    out = _tc_post(s2, q, invd, b2r)
    return (out, h1, out, h1)


# speedup vs baseline: 2.5752x; 1.0180x over previous
import functools

import numpy as np

import jax
import jax.numpy as jnp
from jax import lax
from jax.experimental import pallas as pl
from jax.experimental.pallas import tpu as pltpu
from jax.experimental.pallas import tpu_sc as plsc

N = 10000
E = 320000
D_IN = 128
D2 = 48

NC = 2
NS = 16
NW = NC * NS
CHUNK = 128
NCH_REAL = E // CHUNK
NCH = 2560
CPW = NCH // NW
SUB = 16
NBLK = CPW // SUB
NROW = N + CHUNK
RPT = NROW // NS
DEG_PAD = 10240
RPD = DEG_PAD // NS


def _make_edge_agg(d, with_deg, stage_t, ring):
    mesh = plsc.VectorSubcoreMesh(core_axis_name="c", subcore_axis_name="s")

    out_type = [jax.ShapeDtypeStruct((NC, NROW, d), jnp.float32)]
    if with_deg:
        out_type.append(jax.ShapeDtypeStruct((NC * DEG_PAD,), jnp.float32))

    scratch = [
        pltpu.VMEM((SUB, 2, CHUNK), jnp.int32),
        pltpu.VMEM((ring, CHUNK, d), jnp.float32),
        pltpu.VMEM((CHUNK,), jnp.float32),
        pltpu.VMEM_SHARED((NROW, d), jnp.float32),
        pltpu.VMEM_SHARED((N, d) if stage_t else (8, d), jnp.float32),
        pltpu.VMEM_SHARED((DEG_PAD if with_deg else 16,), jnp.float32),
        pltpu.SemaphoreType.DMA,
        pltpu.SemaphoreType.DMA,
        pltpu.SemaphoreType.DMA,
    ]

    def body(t_hbm, ei_hbm, za_hbm, zd_hbm, out_hbm, deg_hbm,
             idxb_v, rows_v, ones_v, acc, tstage, dacc, sem_s, sem_g, sem_d):
        c = lax.axis_index("c")
        s = lax.axis_index("s")
        w = c * NS + s

        pltpu.sync_copy(za_hbm, acc.at[pl.ds(s * RPT, RPT)])
        if with_deg:
            for i in range(CHUNK // 16):
                ones_v[pl.ds(i * 16, 16)] = jnp.ones((16,), jnp.float32)
            pltpu.sync_copy(zd_hbm, dacc.at[pl.ds(s * RPD, RPD)])
        if stage_t:
            npt = N // NS
            pltpu.sync_copy(t_hbm.at[pl.ds(s * npt, npt)],
                            tstage.at[pl.ds(s * npt, npt)])
        gsrc = tstage if stage_t else t_hbm
        plsc.subcore_barrier()

        def gissue(j):
            pltpu.async_copy(gsrc.at[idxb_v.at[j, 0]],
                             rows_v.at[j % ring], sem_g)

        def gwait():
            pltpu.make_async_copy(gsrc.at[idxb_v.at[0, 0]], rows_v.at[0],
                                  sem_g).wait()

        def swait():
            pltpu.make_async_copy(rows_v.at[0], acc.at[idxb_v.at[0, 1]],
                                  sem_s).wait()

        def dwait():
            pltpu.make_async_copy(ones_v, dacc.at[idxb_v.at[0, 1]],
                                  sem_d).wait()

        chunk0 = w * CPW

        def block(b, carry):
            row0 = chunk0 + b * SUB
            pltpu.sync_copy(ei_hbm.at[pl.ds(row0, SUB)], idxb_v)
            gissue(0)
            gissue(1)

            for j in range(SUB):
                gwait()
                pltpu.async_copy(rows_v.at[j % ring],
                                 acc.at[idxb_v.at[j, 1]], sem_s, add=True)
                if with_deg:
                    pltpu.async_copy(ones_v, dacc.at[idxb_v.at[j, 1]],
                                     sem_d, add=True)
                if j + 2 < SUB:
                    if j >= 1:
                        swait()
                        if with_deg:
                            dwait()
                    gissue(j + 2)

            for _ in range(3):
                swait()
                if with_deg:
                    dwait()
            return carry

        lax.fori_loop(0, NBLK, block, 0)
        plsc.subcore_barrier()
        pltpu.sync_copy(acc.at[pl.ds(s * RPT, RPT)],
                        out_hbm.at[c, pl.ds(s * RPT, RPT)])
        if with_deg:
            pltpu.sync_copy(dacc.at[pl.ds(s * RPD, RPD)],
                            deg_hbm.at[pl.ds(c * DEG_PAD + s * RPD, RPD)])

    if with_deg:
        def agg(t_hbm, ei_hbm, za_hbm, zd_hbm, out_hbm, deg_hbm,
                idxb_v, rows_v, ones_v, acc, tstage, dacc,
                sem_s, sem_g, sem_d):
            body(t_hbm, ei_hbm, za_hbm, zd_hbm, out_hbm, deg_hbm,
                 idxb_v, rows_v, ones_v, acc, tstage, dacc,
                 sem_s, sem_g, sem_d)
    else:
        def agg(t_hbm, ei_hbm, za_hbm, zd_hbm, out_hbm,
                idxb_v, rows_v, ones_v, acc, tstage, dacc,
                sem_s, sem_g, sem_d):
            body(t_hbm, ei_hbm, za_hbm, zd_hbm, out_hbm, None,
                 idxb_v, rows_v, ones_v, acc, tstage, dacc,
                 sem_s, sem_g, sem_d)

    return functools.partial(
        pl.kernel,
        mesh=mesh,
        out_type=out_type if with_deg else out_type[0],
        scratch_types=scratch,
        compiler_params=pltpu.CompilerParams(use_tc_tiling_on_sc=False),
    )(agg)


_agg_d1 = _make_edge_agg(D_IN, with_deg=True, stage_t=False, ring=2)
_agg_d2 = _make_edge_agg(D2, with_deg=False, stage_t=True, ring=3)

_PAD_LANE = np.arange(CHUNK, dtype=np.int32)
_PAD_CHUNKS = np.stack(
    [np.broadcast_to(_PAD_LANE, (NCH - NCH_REAL, CHUNK)),
     np.broadcast_to(N + _PAD_LANE, (NCH - NCH_REAL, CHUNK))],
    axis=1)

_BN = 2000


def _tc_mid(x, p, invd, ws1, b1r, wn1, wn2p, ws2):
    def body(x_ref, p_ref, invd_ref, ws1_ref, b1_ref, wn1_ref, wn2_ref,
             ws2_ref, h1_ref, t2_ref, s2_ref):
        mean = (p_ref[0] + p_ref[1]) * invd_ref[...]
        h1 = jnp.maximum(
            jnp.dot(x_ref[...], ws1_ref[...],
                    preferred_element_type=jnp.float32)
            + jnp.dot(mean, wn1_ref[...], preferred_element_type=jnp.float32)
            + b1_ref[...],
            0.0)
        h1_ref[...] = h1
        t2_ref[...] = jnp.dot(h1, wn2_ref[...],
                              preferred_element_type=jnp.float32)
        s2_ref[...] = jnp.dot(h1, ws2_ref[...],
                              preferred_element_type=jnp.float32)

    return pl.pallas_call(
        body,
        grid=(N // _BN,),
        in_specs=[
            pl.BlockSpec((_BN, D_IN), lambda i: (i, 0)),
            pl.BlockSpec((NC, _BN, D_IN), lambda i: (0, i, 0)),
            pl.BlockSpec((_BN, 1), lambda i: (i, 0)),
            pl.BlockSpec((D_IN, D_IN), lambda i: (0, 0)),
            pl.BlockSpec((1, D_IN), lambda i: (0, 0)),
            pl.BlockSpec((D_IN, D_IN), lambda i: (0, 0)),
            pl.BlockSpec((D_IN, D2), lambda i: (0, 0)),
            pl.BlockSpec((D_IN, 40), lambda i: (0, 0)),
        ],
        out_specs=[
            pl.BlockSpec((_BN, D_IN), lambda i: (i, 0)),
            pl.BlockSpec((_BN, D2), lambda i: (i, 0)),
            pl.BlockSpec((_BN, 40), lambda i: (i, 0)),
        ],
        out_shape=[
            jax.ShapeDtypeStruct((N, D_IN), jnp.float32),
            jax.ShapeDtypeStruct((N, D2), jnp.float32),
            jax.ShapeDtypeStruct((N, 40), jnp.float32),
        ],
    )(x, p, invd, ws1, b1r, wn1, wn2p, ws2)


def _tc_post(s2, q, invd, b2r):
    def body(s2_ref, q_ref, invd_ref, b2_ref, o_ref):
        aggq = q_ref[0] + q_ref[1]
        o_ref[...] = s2_ref[...] + aggq[:, :40] * invd_ref[...] + b2_ref[...]

    return pl.pallas_call(
        body,
        grid=(N // _BN,),
        in_specs=[
            pl.BlockSpec((_BN, 40), lambda i: (i, 0)),
            pl.BlockSpec((NC, _BN, D2), lambda i: (0, i, 0)),
            pl.BlockSpec((_BN, 1), lambda i: (i, 0)),
            pl.BlockSpec((1, 40), lambda i: (0, 0)),
        ],
        out_specs=pl.BlockSpec((_BN, 40), lambda i: (i, 0)),
        out_shape=jax.ShapeDtypeStruct((N, 40), jnp.float32),
    )(s2, q, invd, b2r)


def kernel(inputs, edge_index, W_self1, W_neigh1, b1, W_self2, W_neigh2, b2):
    ei = edge_index.reshape(2, NCH_REAL, CHUNK).transpose(1, 0, 2)
    ei3 = jnp.concatenate([ei, jnp.asarray(_PAD_CHUNKS)], axis=0)

    wn2p = jnp.pad(W_neigh2, ((0, 0), (0, D2 - 40)))
    b1r = b1.reshape(1, D_IN)
    b2r = b2.reshape(1, 40)
    za1 = jnp.zeros((RPT, D_IN), jnp.float32)
    za2 = jnp.zeros((RPT, D2), jnp.float32)
    zd = jnp.zeros((RPD,), jnp.float32)

    p, degs = _agg_d1(inputs, ei3, za1, zd)
    deg = degs[:N] + degs[DEG_PAD:DEG_PAD + N]
    invd = (1.0 / jnp.clip(deg, 1.0, None)).reshape(N, 1)
    h1, t2p, s2 = _tc_mid(inputs, p, invd, W_self1, b1r, W_neigh1,
                          wn2p, W_self2)
    q = _agg_d2(t2p, ei3, za2, zd)
    out = _tc_post(s2, q, invd, b2r)
    return (out, h1, out, h1)


# speedup vs baseline: 2.7214x; 1.0568x over previous
import functools

import numpy as np

import jax
import jax.numpy as jnp
from jax import lax
from jax.experimental import pallas as pl
from jax.experimental.pallas import tpu as pltpu
from jax.experimental.pallas import tpu_sc as plsc

N = 10000
E = 320000
D_IN = 128
D2 = 48

NC = 2
NS = 16
NW = NC * NS
CHUNK = 128
NCH_REAL = E // CHUNK
NCH = 2560
CPW = NCH // NW
SUB = 20
NBLK = CPW // SUB
NROW = N + CHUNK
RPT = NROW // NS
DEG_PAD = 16384
RPD = DEG_PAD // NS


def _make_edge_agg(d, with_deg, stage_t, ring):
    mesh = plsc.VectorSubcoreMesh(core_axis_name="c", subcore_axis_name="s")

    out_type = [jax.ShapeDtypeStruct((NC, NROW, d), jnp.float32)]
    if with_deg:
        out_type.append(jax.ShapeDtypeStruct((NC * DEG_PAD,), jnp.float32))

    scratch = [
        pltpu.VMEM((SUB, 2, CHUNK), jnp.int32),
        pltpu.VMEM((ring, CHUNK, d), jnp.float32),
        pltpu.VMEM((CHUNK,), jnp.float32),
        pltpu.VMEM_SHARED((NROW, d), jnp.float32),
        pltpu.VMEM_SHARED((N, d) if stage_t else (8, d), jnp.float32),
        pltpu.VMEM_SHARED((DEG_PAD if with_deg else 16,), jnp.float32),
        pltpu.SemaphoreType.DMA,
        pltpu.SemaphoreType.DMA,
        pltpu.SemaphoreType.DMA,
    ]

    def body(t_hbm, ei_hbm, pad_hbm, za_hbm, zd_hbm, out_hbm, deg_hbm,
             idxb_v, rows_v, ones_v, acc, tstage, dacc, sem_s, sem_g, sem_d):
        c = lax.axis_index("c")
        s = lax.axis_index("s")
        w = c * NS + s

        pltpu.sync_copy(za_hbm, acc.at[pl.ds(s * RPT, RPT)])
        if with_deg:
            for i in range(CHUNK // 16):
                ones_v[pl.ds(i * 16, 16)] = jnp.ones((16,), jnp.float32)
            pltpu.sync_copy(zd_hbm, dacc.at[pl.ds(s * RPD, RPD)])
        if stage_t:
            npt = N // NS
            pltpu.sync_copy(t_hbm.at[pl.ds(s * npt, npt)],
                            tstage.at[pl.ds(s * npt, npt)])
        gsrc = tstage if stage_t else t_hbm
        plsc.subcore_barrier()

        def gissue(j):
            pltpu.async_copy(gsrc.at[idxb_v.at[j, 0]],
                             rows_v.at[j % ring], sem_g)

        def gwait():
            pltpu.make_async_copy(gsrc.at[idxb_v.at[0, 0]], rows_v.at[0],
                                  sem_g).wait()

        def swait():
            pltpu.make_async_copy(rows_v.at[0], acc.at[idxb_v.at[0, 1]],
                                  sem_s).wait()

        def dwait():
            pltpu.make_async_copy(ones_v, dacc.at[idxb_v.at[0, 1]],
                                  sem_d).wait()

        chunk0 = w * CPW

        def block(b, carry):
            row0 = chunk0 + b * SUB

            @pl.when(row0 < NCH_REAL)
            def _():
                pltpu.sync_copy(ei_hbm.at[pl.ds(row0, SUB)], idxb_v)

            @pl.when(row0 >= NCH_REAL)
            def _():
                pltpu.sync_copy(pad_hbm.at[pl.ds(row0 - NCH_REAL, SUB)],
                                idxb_v)
            gissue(0)
            gissue(1)

            for j in range(SUB):
                gwait()
                pltpu.async_copy(rows_v.at[j % ring],
                                 acc.at[idxb_v.at[j, 1]], sem_s, add=True)
                if with_deg:
                    pltpu.async_copy(ones_v, dacc.at[idxb_v.at[j, 1]],
                                     sem_d, add=True)
                if j + 2 < SUB:
                    if j >= 1:
                        swait()
                        if with_deg:
                            dwait()
                    gissue(j + 2)

            for _ in range(3):
                swait()
                if with_deg:
                    dwait()
            return carry

        lax.fori_loop(0, NBLK, block, 0)
        plsc.subcore_barrier()
        pltpu.sync_copy(acc.at[pl.ds(s * RPT, RPT)],
                        out_hbm.at[c, pl.ds(s * RPT, RPT)])
        if with_deg:
            pltpu.sync_copy(dacc.at[pl.ds(s * RPD, RPD)],
                            deg_hbm.at[pl.ds(c * DEG_PAD + s * RPD, RPD)])

    if with_deg:
        def agg(t_hbm, ei_hbm, pad_hbm, za_hbm, zd_hbm, out_hbm, deg_hbm,
                idxb_v, rows_v, ones_v, acc, tstage, dacc,
                sem_s, sem_g, sem_d):
            body(t_hbm, ei_hbm, pad_hbm, za_hbm, zd_hbm, out_hbm, deg_hbm,
                 idxb_v, rows_v, ones_v, acc, tstage, dacc,
                 sem_s, sem_g, sem_d)
    else:
        def agg(t_hbm, ei_hbm, pad_hbm, za_hbm, zd_hbm, out_hbm,
                idxb_v, rows_v, ones_v, acc, tstage, dacc,
                sem_s, sem_g, sem_d):
            body(t_hbm, ei_hbm, pad_hbm, za_hbm, zd_hbm, out_hbm, None,
                 idxb_v, rows_v, ones_v, acc, tstage, dacc,
                 sem_s, sem_g, sem_d)

    return functools.partial(
        pl.kernel,
        mesh=mesh,
        out_type=out_type if with_deg else out_type[0],
        scratch_types=scratch,
        compiler_params=pltpu.CompilerParams(use_tc_tiling_on_sc=False),
    )(agg)


_agg_d1 = _make_edge_agg(D_IN, with_deg=True, stage_t=False, ring=2)
_agg_d2 = _make_edge_agg(D2, with_deg=False, stage_t=True, ring=3)

_PAD_LANE = np.arange(CHUNK, dtype=np.int32)
_PAD_CHUNKS = np.stack(
    [np.broadcast_to(_PAD_LANE, (NCH - NCH_REAL, CHUNK)),
     np.broadcast_to(N + _PAD_LANE, (NCH - NCH_REAL, CHUNK))],
    axis=1)

_BN = 2048


def _tc_mid(x, p, degs, ws1, b1r, wn1, wn2p, ws2):
    def body(x_ref, p_ref, d0_ref, d1_ref, ws1_ref, b1_ref, wn1_ref,
             wn2_ref, ws2_ref, h1_ref, t2_ref, s2_ref):
        deg = d0_ref[...] + d1_ref[...]
        invd = (1.0 / jnp.maximum(deg, 1.0)).reshape(_BN, 1)
        mean = (p_ref[0] + p_ref[1]) * invd
        h1 = jnp.maximum(
            jnp.dot(x_ref[...], ws1_ref[...],
                    preferred_element_type=jnp.float32)
            + jnp.dot(mean, wn1_ref[...], preferred_element_type=jnp.float32)
            + b1_ref[...],
            0.0)
        h1_ref[...] = h1
        t2_ref[...] = jnp.dot(h1, wn2_ref[...],
                              preferred_element_type=jnp.float32)
        s2_ref[...] = jnp.dot(h1, ws2_ref[...],
                              preferred_element_type=jnp.float32)

    return pl.pallas_call(
        body,
        grid=(-(-N // _BN),),
        in_specs=[
            pl.BlockSpec((_BN, D_IN), lambda i: (i, 0)),
            pl.BlockSpec((NC, _BN, D_IN), lambda i: (0, i, 0)),
            pl.BlockSpec((_BN,), lambda i: (i,)),
            pl.BlockSpec((_BN,), lambda i: (i + DEG_PAD // _BN,)),
            pl.BlockSpec((D_IN, D_IN), lambda i: (0, 0)),
            pl.BlockSpec((1, D_IN), lambda i: (0, 0)),
            pl.BlockSpec((D_IN, D_IN), lambda i: (0, 0)),
            pl.BlockSpec((D_IN, D2), lambda i: (0, 0)),
            pl.BlockSpec((D_IN, 40), lambda i: (0, 0)),
        ],
        out_specs=[
            pl.BlockSpec((_BN, D_IN), lambda i: (i, 0)),
            pl.BlockSpec((_BN, D2), lambda i: (i, 0)),
            pl.BlockSpec((_BN, 40), lambda i: (i, 0)),
        ],
        out_shape=[
            jax.ShapeDtypeStruct((N, D_IN), jnp.float32),
            jax.ShapeDtypeStruct((N, D2), jnp.float32),
            jax.ShapeDtypeStruct((N, 40), jnp.float32),
        ],
    )(x, p, degs, degs, ws1, b1r, wn1, wn2p, ws2)


def _tc_post(s2, q, degs, b2r):
    def body(s2_ref, q_ref, d0_ref, d1_ref, b2_ref, o_ref):
        deg = d0_ref[...] + d1_ref[...]
        invd = (1.0 / jnp.maximum(deg, 1.0)).reshape(_BN, 1)
        aggq = q_ref[0] + q_ref[1]
        o_ref[...] = s2_ref[...] + aggq[:, :40] * invd + b2_ref[...]

    return pl.pallas_call(
        body,
        grid=(-(-N // _BN),),
        in_specs=[
            pl.BlockSpec((_BN, 40), lambda i: (i, 0)),
            pl.BlockSpec((NC, _BN, D2), lambda i: (0, i, 0)),
            pl.BlockSpec((_BN,), lambda i: (i,)),
            pl.BlockSpec((_BN,), lambda i: (i + DEG_PAD // _BN,)),
            pl.BlockSpec((1, 40), lambda i: (0, 0)),
        ],
        out_specs=pl.BlockSpec((_BN, 40), lambda i: (i, 0)),
        out_shape=jax.ShapeDtypeStruct((N, 40), jnp.float32),
    )(s2, q, degs, degs, b2r)


def kernel(inputs, edge_index, W_self1, W_neigh1, b1, W_self2, W_neigh2, b2):
    ei = edge_index.reshape(2, NCH_REAL, CHUNK).transpose(1, 0, 2)
    pad = jnp.asarray(_PAD_CHUNKS)

    wn2p = jnp.pad(W_neigh2, ((0, 0), (0, D2 - 40)))
    b1r = b1.reshape(1, D_IN)
    b2r = b2.reshape(1, 40)
    za1 = jnp.zeros((RPT, D_IN), jnp.float32)
    za2 = jnp.zeros((RPT, D2), jnp.float32)
    zd = jnp.zeros((RPD,), jnp.float32)

    p, degs = _agg_d1(inputs, ei, pad, za1, zd)
    h1, t2p, s2 = _tc_mid(inputs, p, degs, W_self1, b1r, W_neigh1,
                          wn2p, W_self2)
    q = _agg_d2(t2p, ei, pad, za2, zd)
    out = _tc_post(s2, q, degs, b2r)
    return (out, h1, out, h1)
